# Initial kernel scaffold; baseline (speedup 1.0000x reference)
import jax
import jax.numpy as jnp
from jax.experimental import pallas as pl


def kernel(center, x, affine_alpha, affine_beta, ln1_g, ln1_b, attn_w1, attn_b1, attn_w2, attn_b2, ln2_g, ln2_b, mlp_w1, mlp_b1, mlp_w2, mlp_b2):
    raise NotImplementedError("write your pallas kernel here")


# speedup vs baseline: 17.6354x; 17.6354x over previous
import math

import jax
import jax.numpy as jnp
from jax.experimental import pallas as pl
from jax.experimental.pallas import tpu as pltpu

_K = 16
_T = 256


def _ln(x, g, b, eps=1e-5):
    m = jnp.mean(x, axis=-1, keepdims=True)
    v = jnp.mean((x - m) ** 2, axis=-1, keepdims=True)
    return (x - m) / jnp.sqrt(v + eps) * g + b


def _gelu(x):
    return 0.5 * x * (1.0 + jax.lax.erf(x / math.sqrt(2.0)))


def _agg_kernel(ctr_tile_ref, ctr_t_ref, x_full_ref, x_tile_ref, g_ref, b_ref,
                t1_ref, sums_ref):
    b = pl.program_id(0)
    t = pl.program_id(1)
    ct = ctr_tile_ref[0]
    xcT = ctr_t_ref[0]
    sqt = jnp.sum(ct * ct, axis=1, keepdims=True)
    sqf = jnp.sum(xcT * xcT, axis=0, keepdims=True)
    cross = jax.lax.dot_general(ct, xcT, (((1,), (0,)), ((), ())),
                                preferred_element_type=jnp.float32)
    d2 = sqt + sqf - 2.0 * cross

    dm = d2
    sel = jnp.zeros_like(d2)
    for _ in range(_K):
        rowmin = jnp.min(dm, axis=1, keepdims=True)
        hit = dm <= rowmin
        sel = jnp.where(hit, 1.0, sel)
        dm = jnp.where(hit, jnp.float32(jnp.inf), dm)

    g = g_ref[...]
    bb = b_ref[...]
    xn = _ln(x_full_ref[0], g, bb)
    q = jnp.sum(xn * xn, axis=1, keepdims=True)
    lane = jax.lax.broadcasted_iota(jnp.int32, (xn.shape[0], 128), 1)
    qcol = jnp.where(lane == 0, q, 0.0)
    xe = jnp.concatenate([xn, qcol], axis=1)
    t1e = jax.lax.dot_general(sel, xe, (((1,), (0,)), ((), ())),
                              preferred_element_type=jnp.float32)
    t1 = t1e[:, :128]
    tq = t1e[:, 128:129]

    xt = _ln(x_tile_ref[0], g, bb)
    qt = jnp.sum(xt * xt, axis=1, keepdims=True)
    psy = jnp.sum(t1) - _K * jnp.sum(xt)
    psy2 = jnp.sum(tq) + _K * jnp.sum(qt) - 2.0 * jnp.sum(xt * t1)

    t1_ref[0] = t1

    @pl.when(jnp.logical_and(b == 0, t == 0))
    def _():
        sums_ref[0] = 0.0
        sums_ref[1] = 0.0

    sums_ref[0] += psy
    sums_ref[1] += psy2


def _mlp_kernel(t1_ref, x_tile_ref, xcls_ref, scal_ref,
                g1_ref, b1_ref, g2_ref, b2_ref, al_ref, be_ref,
                aw1_ref, ab1_ref, aw2_ref, ab2_ref,
                mw1_ref, mb1_ref, mw2_ref, mb2_ref,
                out_ref, outc_ref):
    b = pl.program_id(0)
    t = pl.program_id(1)
    stdinv = scal_ref[0]
    g1 = g1_ref[...]
    bb1 = b1_ref[...]
    g2 = g2_ref[...]
    bb2 = b2_ref[...]

    xt = x_tile_ref[0]
    xn = _ln(xt, g1, bb1)
    s = t1_ref[0] * (1.0 / _K)
    e1 = (s - xn) * stdinv
    enh = jnp.concatenate([e1, xn], axis=1)
    enh = enh * al_ref[...] + be_ref[...]
    h = _gelu(jnp.dot(enh, aw1_ref[...],
                      preferred_element_type=jnp.float32) + ab1_ref[...])
    a = jnp.dot(h, aw2_ref[...],
                preferred_element_type=jnp.float32) + ab2_ref[...]
    y = xt + a
    y2 = _ln(y, g2, bb2)
    m = _gelu(jnp.dot(y2, mw1_ref[...],
                      preferred_element_type=jnp.float32) + mb1_ref[...])
    out_ref[0] = y + jnp.dot(m, mw2_ref[...],
                             preferred_element_type=jnp.float32) + mb2_ref[...]

    @pl.when(jnp.logical_and(b == 0, t == 0))
    def _():
        xc = xcls_ref[...]
        c0 = xc + _ln(xc, g1, bb1)
        y2c = _ln(c0, g2, bb2)
        mc = _gelu(jnp.dot(y2c, mw1_ref[...],
                           preferred_element_type=jnp.float32) + mb1_ref[...])
        outc_ref[...] = c0 + jnp.dot(mc, mw2_ref[...],
                                     preferred_element_type=jnp.float32) + mb2_ref[...]


def kernel(center, x, affine_alpha, affine_beta, ln1_g, ln1_b,
           attn_w1, attn_b1, attn_w2, attn_b2, ln2_g, ln2_b,
           mlp_w1, mlp_b1, mlp_w2, mlp_b2):
    B, Np1, C = x.shape
    G = Np1 - 1
    nT = G // _T
    H = mlp_w1.shape[1]

    x_cls = x[:, 0, :]
    x_rest = x[:, 1:, :]
    ctr_t = jnp.transpose(center, (0, 2, 1))
    g1 = ln1_g.reshape(1, C)
    b1 = ln1_b.reshape(1, C)
    g2 = ln2_g.reshape(1, C)
    b2 = ln2_b.reshape(1, C)

    t1, sums = pl.pallas_call(
        _agg_kernel,
        grid=(B, nT),
        in_specs=[
            pl.BlockSpec((1, _T, 3), lambda b, t: (b, t, 0)),
            pl.BlockSpec((1, 3, G), lambda b, t: (b, 0, 0)),
            pl.BlockSpec((1, G, C), lambda b, t: (b, 0, 0)),
            pl.BlockSpec((1, _T, C), lambda b, t: (b, t, 0)),
            pl.BlockSpec((1, C), lambda b, t: (0, 0)),
            pl.BlockSpec((1, C), lambda b, t: (0, 0)),
        ],
        out_specs=[
            pl.BlockSpec((1, _T, C), lambda b, t: (b, t, 0)),
            pl.BlockSpec(memory_space=pltpu.SMEM),
        ],
        out_shape=[
            jax.ShapeDtypeStruct((B, G, C), jnp.float32),
            jax.ShapeDtypeStruct((2,), jnp.float32),
        ],
    )(center, ctr_t, x_rest, x_rest, g1, b1)

    M = B * G * _K * C
    var = (sums[1] - sums[0] * sums[0] / M) / (M - 1)
    stdinv = (1.0 / (jnp.sqrt(var) + 1e-05)).reshape(1)

    out_rest, out_cls = pl.pallas_call(
        _mlp_kernel,
        grid=(B, nT),
        in_specs=[
            pl.BlockSpec((1, _T, C), lambda b, t: (b, t, 0)),
            pl.BlockSpec((1, _T, C), lambda b, t: (b, t, 0)),
            pl.BlockSpec((B, C), lambda b, t: (0, 0)),
            pl.BlockSpec(memory_space=pltpu.SMEM),
            pl.BlockSpec((1, C), lambda b, t: (0, 0)),
            pl.BlockSpec((1, C), lambda b, t: (0, 0)),
            pl.BlockSpec((1, C), lambda b, t: (0, 0)),
            pl.BlockSpec((1, C), lambda b, t: (0, 0)),
            pl.BlockSpec((1, 2 * C), lambda b, t: (0, 0)),
            pl.BlockSpec((1, 2 * C), lambda b, t: (0, 0)),
            pl.BlockSpec((2 * C, C), lambda b, t: (0, 0)),
            pl.BlockSpec((1, C), lambda b, t: (0, 0)),
            pl.BlockSpec((C, C), lambda b, t: (0, 0)),
            pl.BlockSpec((1, C), lambda b, t: (0, 0)),
            pl.BlockSpec((C, H), lambda b, t: (0, 0)),
            pl.BlockSpec((1, H), lambda b, t: (0, 0)),
            pl.BlockSpec((H, C), lambda b, t: (0, 0)),
            pl.BlockSpec((1, C), lambda b, t: (0, 0)),
        ],
        out_specs=[
            pl.BlockSpec((1, _T, C), lambda b, t: (b, t, 0)),
            pl.BlockSpec((B, C), lambda b, t: (0, 0)),
        ],
        out_shape=[
            jax.ShapeDtypeStruct((B, G, C), jnp.float32),
            jax.ShapeDtypeStruct((B, C), jnp.float32),
        ],
    )(t1, x_rest, x_cls, stdinv, g1, b1, g2, b2,
      affine_alpha.reshape(1, 2 * C), affine_beta.reshape(1, 2 * C),
      attn_w1, attn_b1.reshape(1, C), attn_w2, attn_b2.reshape(1, C),
      mlp_w1, mlp_b1.reshape(1, H), mlp_w2, mlp_b2.reshape(1, C))

    return jnp.concatenate([out_cls[:, None, :], out_rest], axis=1)


# speedup vs baseline: 25.2722x; 1.4330x over previous
import math

import jax
import jax.numpy as jnp
from jax.experimental import pallas as pl
from jax.experimental.pallas import tpu as pltpu

_K = 16
_T = 512


def _ln(x, g, b, eps=1e-5):
    m = jnp.mean(x, axis=-1, keepdims=True)
    v = jnp.mean((x - m) ** 2, axis=-1, keepdims=True)
    return (x - m) / jnp.sqrt(v + eps) * g + b


def _gelu(x):
    return 0.5 * x * (1.0 + jax.lax.erf(x / math.sqrt(2.0)))


def _agg_kernel(ctr_tile_ref, ctr_t_ref, x_full_ref, x_tile_ref, g_ref, b_ref,
                t1_ref, sums_ref):
    b = pl.program_id(0)
    t = pl.program_id(1)
    ct = ctr_tile_ref[0]
    xcT = ctr_t_ref[0]
    sqt = jnp.sum(ct * ct, axis=1, keepdims=True)
    sqf = jnp.sum(xcT * xcT, axis=0, keepdims=True)
    cross = jax.lax.dot_general(ct, xcT, (((1,), (0,)), ((), ())),
                                preferred_element_type=jnp.float32)
    d2 = sqt + sqf - 2.0 * cross

    dm = d2
    for _ in range(_K):
        rowmin = jnp.min(dm, axis=1, keepdims=True)
        dm = jnp.where(dm <= rowmin, jnp.float32(jnp.inf), dm)
    sel = jnp.where(jnp.isinf(dm), 1.0, 0.0)

    g = g_ref[...]
    bb = b_ref[...]
    xn = _ln(x_full_ref[0], g, bb)
    q = jnp.sum(xn * xn, axis=1, keepdims=True)
    lane = jax.lax.broadcasted_iota(jnp.int32, (xn.shape[0], 128), 1)
    qcol = jnp.where(lane == 0, q, 0.0)
    xe = jnp.concatenate([xn, qcol], axis=1)
    t1e = jax.lax.dot_general(sel, xe, (((1,), (0,)), ((), ())),
                              preferred_element_type=jnp.float32)
    t1 = t1e[:, :128]
    tq = t1e[:, 128:129]

    xt = _ln(x_tile_ref[0], g, bb)
    qt = jnp.sum(xt * xt, axis=1, keepdims=True)
    psy = jnp.sum(t1) - _K * jnp.sum(xt)
    psy2 = jnp.sum(tq) + _K * jnp.sum(qt) - 2.0 * jnp.sum(xt * t1)

    t1_ref[0] = t1

    @pl.when(jnp.logical_and(b == 0, t == 0))
    def _():
        sums_ref[0] = 0.0
        sums_ref[1] = 0.0

    sums_ref[0] += psy
    sums_ref[1] += psy2


def _mlp_kernel(t1_ref, x_tile_ref, xcls_ref, scal_ref,
                g1_ref, b1_ref, g2_ref, b2_ref, al_ref, be_ref,
                aw1_ref, ab1_ref, aw2_ref, ab2_ref,
                mw1_ref, mb1_ref, mw2_ref, mb2_ref,
                out_ref, outc_ref):
    b = pl.program_id(0)
    t = pl.program_id(1)
    stdinv = scal_ref[0]
    g1 = g1_ref[...]
    bb1 = b1_ref[...]
    g2 = g2_ref[...]
    bb2 = b2_ref[...]

    xt = x_tile_ref[0]
    xn = _ln(xt, g1, bb1)
    s = t1_ref[0] * (1.0 / _K)
    e1 = (s - xn) * stdinv
    enh = jnp.concatenate([e1, xn], axis=1)
    enh = enh * al_ref[...] + be_ref[...]
    h = _gelu(jnp.dot(enh, aw1_ref[...],
                      preferred_element_type=jnp.float32) + ab1_ref[...])
    a = jnp.dot(h, aw2_ref[...],
                preferred_element_type=jnp.float32) + ab2_ref[...]
    y = xt + a
    y2 = _ln(y, g2, bb2)
    m = _gelu(jnp.dot(y2, mw1_ref[...],
                      preferred_element_type=jnp.float32) + mb1_ref[...])
    out_ref[0] = y + jnp.dot(m, mw2_ref[...],
                             preferred_element_type=jnp.float32) + mb2_ref[...]

    @pl.when(jnp.logical_and(b == 0, t == 0))
    def _():
        xc = xcls_ref[...]
        c0 = xc + _ln(xc, g1, bb1)
        y2c = _ln(c0, g2, bb2)
        mc = _gelu(jnp.dot(y2c, mw1_ref[...],
                           preferred_element_type=jnp.float32) + mb1_ref[...])
        outc_ref[...] = c0 + jnp.dot(mc, mw2_ref[...],
                                     preferred_element_type=jnp.float32) + mb2_ref[...]


def kernel(center, x, affine_alpha, affine_beta, ln1_g, ln1_b,
           attn_w1, attn_b1, attn_w2, attn_b2, ln2_g, ln2_b,
           mlp_w1, mlp_b1, mlp_w2, mlp_b2):
    B, Np1, C = x.shape
    G = Np1 - 1
    nT = G // _T
    H = mlp_w1.shape[1]

    x_cls = x[:, 0, :]
    x_rest = x[:, 1:, :]
    ctr_t = jnp.transpose(center, (0, 2, 1))
    g1 = ln1_g.reshape(1, C)
    b1 = ln1_b.reshape(1, C)
    g2 = ln2_g.reshape(1, C)
    b2 = ln2_b.reshape(1, C)

    t1, sums = pl.pallas_call(
        _agg_kernel,
        grid=(B, nT),
        in_specs=[
            pl.BlockSpec((1, _T, 3), lambda b, t: (b, t, 0)),
            pl.BlockSpec((1, 3, G), lambda b, t: (b, 0, 0)),
            pl.BlockSpec((1, G, C), lambda b, t: (b, 0, 0)),
            pl.BlockSpec((1, _T, C), lambda b, t: (b, t, 0)),
            pl.BlockSpec((1, C), lambda b, t: (0, 0)),
            pl.BlockSpec((1, C), lambda b, t: (0, 0)),
        ],
        out_specs=[
            pl.BlockSpec((1, _T, C), lambda b, t: (b, t, 0)),
            pl.BlockSpec(memory_space=pltpu.SMEM),
        ],
        out_shape=[
            jax.ShapeDtypeStruct((B, G, C), jnp.float32),
            jax.ShapeDtypeStruct((2,), jnp.float32),
        ],
    )(center, ctr_t, x_rest, x_rest, g1, b1)

    M = B * G * _K * C
    var = (sums[1] - sums[0] * sums[0] / M) / (M - 1)
    stdinv = (1.0 / (jnp.sqrt(var) + 1e-05)).reshape(1)

    out_rest, out_cls = pl.pallas_call(
        _mlp_kernel,
        grid=(B, nT),
        in_specs=[
            pl.BlockSpec((1, _T, C), lambda b, t: (b, t, 0)),
            pl.BlockSpec((1, _T, C), lambda b, t: (b, t, 0)),
            pl.BlockSpec((B, C), lambda b, t: (0, 0)),
            pl.BlockSpec(memory_space=pltpu.SMEM),
            pl.BlockSpec((1, C), lambda b, t: (0, 0)),
            pl.BlockSpec((1, C), lambda b, t: (0, 0)),
            pl.BlockSpec((1, C), lambda b, t: (0, 0)),
            pl.BlockSpec((1, C), lambda b, t: (0, 0)),
            pl.BlockSpec((1, 2 * C), lambda b, t: (0, 0)),
            pl.BlockSpec((1, 2 * C), lambda b, t: (0, 0)),
            pl.BlockSpec((2 * C, C), lambda b, t: (0, 0)),
            pl.BlockSpec((1, C), lambda b, t: (0, 0)),
            pl.BlockSpec((C, C), lambda b, t: (0, 0)),
            pl.BlockSpec((1, C), lambda b, t: (0, 0)),
            pl.BlockSpec((C, H), lambda b, t: (0, 0)),
            pl.BlockSpec((1, H), lambda b, t: (0, 0)),
            pl.BlockSpec((H, C), lambda b, t: (0, 0)),
            pl.BlockSpec((1, C), lambda b, t: (0, 0)),
        ],
        out_specs=[
            pl.BlockSpec((1, _T, C), lambda b, t: (b, t, 0)),
            pl.BlockSpec((B, C), lambda b, t: (0, 0)),
        ],
        out_shape=[
            jax.ShapeDtypeStruct((B, G, C), jnp.float32),
            jax.ShapeDtypeStruct((B, C), jnp.float32),
        ],
    )(t1, x_rest, x_cls, stdinv, g1, b1, g2, b2,
      affine_alpha.reshape(1, 2 * C), affine_beta.reshape(1, 2 * C),
      attn_w1, attn_b1.reshape(1, C), attn_w2, attn_b2.reshape(1, C),
      mlp_w1, mlp_b1.reshape(1, H), mlp_w2, mlp_b2.reshape(1, C))

    return jnp.concatenate([out_cls[:, None, :], out_rest], axis=1)
